# Initial kernel scaffold; baseline (speedup 1.0000x reference)
import jax
import jax.numpy as jnp
from jax.experimental import pallas as pl


def kernel(h, w, Wv0, bv0, Wo0, bo0, Wv1, bv1, Wo1, bo1, Wg, bg):
    raise NotImplementedError("write your pallas kernel here")


# speedup vs baseline: 6.9367x; 6.9367x over previous
import functools
import jax
import jax.numpy as jnp
from jax.experimental import pallas as pl

N = 4096
D = 512
TOPK = 32
NEG = float("-inf")


def _topk_norm_body(w_ref, out_ref):
    w = w_ref[...]

    def step(_, t):
        masked = jnp.where(w < t, w, NEG)
        return jnp.max(masked, axis=1, keepdims=True)

    t = jax.lax.fori_loop(0, TOPK, step, jnp.full((w.shape[0], 1), jnp.inf, jnp.float32))
    wsp = jnp.where(w >= t, w, 0.0)
    rs = jnp.sum(wsp, axis=1, keepdims=True) + 1e-8
    out_ref[...] = wsp / rs


def _vproj_body(h_ref, Wv_ref, bv_ref, out_ref):
    out_ref[...] = (
        jnp.dot(h_ref[...], Wv_ref[...], preferred_element_type=jnp.float32)
        + bv_ref[...]
    )


def _msg_body(wn_ref, V_ref, out_ref):
    out_ref[...] = jnp.dot(wn_ref[...], V_ref[...], preferred_element_type=jnp.float32)


def _gate_body(h_ref, msg_ref, Wo_ref, bo_ref, Wg_ref, bg_ref, out_ref):
    h = h_ref[...]
    msg = msg_ref[...]
    out = jnp.dot(msg, Wo_ref[...], preferred_element_type=jnp.float32) + bo_ref[...]
    alpha = jax.nn.sigmoid(
        jnp.dot(h, Wg_ref[...], preferred_element_type=jnp.float32) + bg_ref[...]
    )
    out_ref[...] = alpha * h + (1.0 - alpha) * out


@jax.jit
def kernel(h, w, Wv0, bv0, Wo0, bo0, Wv1, bv1, Wo1, bo1, Wg, bg):
    BR = 512
    BM = 256

    w_norm = pl.pallas_call(
        _topk_norm_body,
        grid=(N // BR,),
        in_specs=[pl.BlockSpec((BR, N), lambda i: (i, 0))],
        out_specs=pl.BlockSpec((BR, N), lambda i: (i, 0)),
        out_shape=jax.ShapeDtypeStruct((N, N), jnp.float32),
    )(w)

    vproj = pl.pallas_call(
        _vproj_body,
        grid=(N // BR,),
        in_specs=[
            pl.BlockSpec((BR, D), lambda i: (i, 0)),
            pl.BlockSpec((D, D), lambda i: (0, 0)),
            pl.BlockSpec((1, D), lambda i: (0, 0)),
        ],
        out_specs=pl.BlockSpec((BR, D), lambda i: (i, 0)),
        out_shape=jax.ShapeDtypeStruct((N, D), jnp.float32),
    )

    msg_mm = pl.pallas_call(
        _msg_body,
        grid=(N // BM,),
        in_specs=[
            pl.BlockSpec((BM, N), lambda i: (i, 0)),
            pl.BlockSpec((N, D), lambda i: (0, 0)),
        ],
        out_specs=pl.BlockSpec((BM, D), lambda i: (i, 0)),
        out_shape=jax.ShapeDtypeStruct((N, D), jnp.float32),
    )

    gate = pl.pallas_call(
        _gate_body,
        grid=(N // BR,),
        in_specs=[
            pl.BlockSpec((BR, D), lambda i: (i, 0)),
            pl.BlockSpec((BR, D), lambda i: (i, 0)),
            pl.BlockSpec((D, D), lambda i: (0, 0)),
            pl.BlockSpec((1, D), lambda i: (0, 0)),
            pl.BlockSpec((D, 1), lambda i: (0, 0)),
            pl.BlockSpec((1, 1), lambda i: (0, 0)),
        ],
        out_specs=pl.BlockSpec((BR, D), lambda i: (i, 0)),
        out_shape=jax.ShapeDtypeStruct((N, D), jnp.float32),
    )

    bg2 = bg.reshape(1, 1)
    for (Wv, bv, Wo, bo) in ((Wv0, bv0, Wo0, bo0), (Wv1, bv1, Wo1, bo1)):
        V = vproj(h, Wv, bv.reshape(1, D))
        msg = msg_mm(w_norm, V)
        h = gate(h, msg, Wo, bo.reshape(1, D), Wg, bg2)
    return h
